# baseline (speedup 1.0000x reference)
import functools

import jax
import jax.numpy as jnp
from jax import lax
from jax.experimental import pallas as pl
from jax.experimental.pallas import tpu as pltpu

NUM_EXPERTS = 64
TOP_K = 8
SEQ = 16384

_ROT = ((13, 15, 26, 6), (17, 29, 16, 24))
_U32 = jnp.uint32


def _rotl(x, r):
    return (x << _U32(r)) | (x >> _U32(32 - r))


def _threefry_xored(k0, k1, x1):
    ks2 = _U32(0x1BD11BDA) ^ k0 ^ k1
    ks = (k0, k1, ks2)
    x0 = jnp.zeros_like(x1) + k0
    x1 = x1 + k1
    for d in range(5):
        rr = _ROT[d % 2]
        for i in range(4):
            x0 = x0 + x1
            x1 = _rotl(x1, rr[i])
            x1 = x1 ^ x0
        x0 = x0 + ks[(d + 1) % 3]
        x1 = x1 + ks[(d + 2) % 3] + _U32(d + 1)
    return x0 ^ x1


def _bits_to_normal(bits):
    flt = lax.bitcast_convert_type(
        (bits >> _U32(9)) | _U32(0x3F800000), jnp.float32
    ) - jnp.float32(1.0)
    lo = jnp.float32(-0.99999994)
    hi = jnp.float32(1.0)
    u = jnp.maximum(lo, flt * (hi - lo) + lo)
    return jnp.float32(1.4142135623730951) * lax.erf_inv(u)


def _router_body(rows_per_blk, kd_ref, x_ref, xout_ref,
                 probs_ref, topv_ref, topi_ref):
    i = pl.program_id(0)
    k0 = kd_ref[0]
    k1 = kd_ref[1]
    xout_ref[...] = x_ref[...]
    base = (i * (rows_per_blk * NUM_EXPERTS)).astype(_U32)
    row = lax.broadcasted_iota(_U32, (rows_per_blk, NUM_EXPERTS), 0)
    col = lax.broadcasted_iota(_U32, (rows_per_blk, NUM_EXPERTS), 1)
    cnt = base + row * _U32(NUM_EXPERTS) + col
    bits = _threefry_xored(k0, k1, cnt)
    probs = _bits_to_normal(bits)
    probs_ref[...] = probs

    cidx = lax.broadcasted_iota(jnp.int32, (rows_per_blk, NUM_EXPERTS), 1)
    vals = probs
    neg = jnp.float32(-jnp.inf)
    for j in range(TOP_K):
        m = jnp.max(vals, axis=1, keepdims=True)
        hit = vals == m
        idx = jnp.min(jnp.where(hit, cidx, NUM_EXPERTS), axis=1, keepdims=True)
        topv_ref[:, j : j + 1] = m
        topi_ref[:, j : j + 1] = idx
        vals = jnp.where(cidx == idx, neg, vals)


def kernel(x, key):
    kd = jax.random.key_data(key).astype(jnp.uint32)
    rows_per_blk = 256
    grid = (SEQ // rows_per_blk,)
    d_model = x.shape[1]
    xout, probs, topv, topi = pl.pallas_call(
        functools.partial(_router_body, rows_per_blk),
        grid=grid,
        in_specs=[
            pl.BlockSpec(memory_space=pltpu.SMEM),
            pl.BlockSpec((rows_per_blk, d_model), lambda i: (i, 0)),
        ],
        out_specs=[
            pl.BlockSpec((rows_per_blk, d_model), lambda i: (i, 0)),
            pl.BlockSpec((rows_per_blk, NUM_EXPERTS), lambda i: (i, 0)),
            pl.BlockSpec((rows_per_blk, TOP_K), lambda i: (i, 0)),
            pl.BlockSpec((rows_per_blk, TOP_K), lambda i: (i, 0)),
        ],
        out_shape=[
            jax.ShapeDtypeStruct(x.shape, x.dtype),
            jax.ShapeDtypeStruct((SEQ, NUM_EXPERTS), jnp.float32),
            jax.ShapeDtypeStruct((SEQ, TOP_K), jnp.float32),
            jax.ShapeDtypeStruct((SEQ, TOP_K), jnp.int32),
        ],
        compiler_params=pltpu.CompilerParams(
            dimension_semantics=("arbitrary",),
        ),
    )(kd, x)
    return (xout, probs, topv, topi)


# speedup vs baseline: 1.1911x; 1.1911x over previous
import functools

import jax
import jax.numpy as jnp
from jax import lax
from jax.experimental import pallas as pl
from jax.experimental.pallas import tpu as pltpu

NUM_EXPERTS = 64
TOP_K = 8
SEQ = 16384

_ROT = ((13, 15, 26, 6), (17, 29, 16, 24))
_U32 = jnp.uint32


def _rotl(x, r):
    return (x << _U32(r)) | (x >> _U32(32 - r))


def _threefry_xored(k0, k1, x1):
    ks2 = _U32(0x1BD11BDA) ^ k0 ^ k1
    ks = (k0, k1, ks2)
    x0 = jnp.zeros_like(x1) + k0
    x1 = x1 + k1
    for d in range(5):
        rr = _ROT[d % 2]
        for i in range(4):
            x0 = x0 + x1
            x1 = _rotl(x1, rr[i])
            x1 = x1 ^ x0
        x0 = x0 + ks[(d + 1) % 3]
        x1 = x1 + ks[(d + 2) % 3] + _U32(d + 1)
    return x0 ^ x1


def _bits_to_normal(bits):
    flt = lax.bitcast_convert_type(
        (bits >> _U32(9)) | _U32(0x3F800000), jnp.float32
    ) - jnp.float32(1.0)
    lo = jnp.float32(-0.99999994)
    hi = jnp.float32(1.0)
    u = jnp.maximum(lo, flt * (hi - lo) + lo)
    return jnp.float32(1.4142135623730951) * lax.erf_inv(u)


def _router_body(rows_per_blk, kd_ref, x_ref, xout_ref,
                 probs_ref, topv_ref, topi_ref):
    i = pl.program_id(0)
    k0 = kd_ref[0]
    k1 = kd_ref[1]
    xout_ref[...] = x_ref[...]
    base = (i * (rows_per_blk * NUM_EXPERTS)).astype(_U32)
    row = lax.broadcasted_iota(_U32, (rows_per_blk, NUM_EXPERTS), 0)
    col = lax.broadcasted_iota(_U32, (rows_per_blk, NUM_EXPERTS), 1)
    cnt = base + row * _U32(NUM_EXPERTS) + col
    bits = _threefry_xored(k0, k1, cnt)
    probs = _bits_to_normal(bits)
    probs_ref[...] = probs

    cidx = lax.broadcasted_iota(jnp.int32, (rows_per_blk, NUM_EXPERTS), 1)
    vals = probs
    neg = jnp.float32(-jnp.inf)
    for j in range(TOP_K):
        m = jnp.max(vals, axis=1, keepdims=True)
        hit = vals == m
        idx = jnp.min(jnp.where(hit, cidx, NUM_EXPERTS), axis=1, keepdims=True)
        topv_ref[:, j : j + 1] = m
        topi_ref[:, j : j + 1] = idx
        vals = jnp.where(cidx == idx, neg, vals)


def kernel(x, key):
    kd = jax.random.key_data(key).astype(jnp.uint32)
    rows_per_blk = 512
    grid = (SEQ // rows_per_blk,)
    d_model = x.shape[1]
    xout, probs, topv, topi = pl.pallas_call(
        functools.partial(_router_body, rows_per_blk),
        grid=grid,
        in_specs=[
            pl.BlockSpec(memory_space=pltpu.SMEM),
            pl.BlockSpec((rows_per_blk, d_model), lambda i: (i, 0)),
        ],
        out_specs=[
            pl.BlockSpec((rows_per_blk, d_model), lambda i: (i, 0)),
            pl.BlockSpec((rows_per_blk, NUM_EXPERTS), lambda i: (i, 0)),
            pl.BlockSpec((rows_per_blk, TOP_K), lambda i: (i, 0)),
            pl.BlockSpec((rows_per_blk, TOP_K), lambda i: (i, 0)),
        ],
        out_shape=[
            jax.ShapeDtypeStruct(x.shape, x.dtype),
            jax.ShapeDtypeStruct((SEQ, NUM_EXPERTS), jnp.float32),
            jax.ShapeDtypeStruct((SEQ, TOP_K), jnp.float32),
            jax.ShapeDtypeStruct((SEQ, TOP_K), jnp.int32),
        ],
        compiler_params=pltpu.CompilerParams(
            dimension_semantics=("arbitrary",),
        ),
    )(kd, x)
    return (xout, probs, topv, topi)


# speedup vs baseline: 1.2058x; 1.0123x over previous
import functools

import jax
import jax.numpy as jnp
from jax import lax
from jax.experimental import pallas as pl
from jax.experimental.pallas import tpu as pltpu

NUM_EXPERTS = 64
TOP_K = 8
SEQ = 16384

_ROT = ((13, 15, 26, 6), (17, 29, 16, 24))
_U32 = jnp.uint32


def _rotl(x, r):
    return (x << _U32(r)) | (x >> _U32(32 - r))


def _threefry_xored(k0, k1, x1):
    ks2 = _U32(0x1BD11BDA) ^ k0 ^ k1
    ks = (k0, k1, ks2)
    x0 = jnp.zeros_like(x1) + k0
    x1 = x1 + k1
    for d in range(5):
        rr = _ROT[d % 2]
        for i in range(4):
            x0 = x0 + x1
            x1 = _rotl(x1, rr[i])
            x1 = x1 ^ x0
        x0 = x0 + ks[(d + 1) % 3]
        x1 = x1 + ks[(d + 2) % 3] + _U32(d + 1)
    return x0 ^ x1


def _bits_to_normal(bits):
    flt = lax.bitcast_convert_type(
        (bits >> _U32(9)) | _U32(0x3F800000), jnp.float32
    ) - jnp.float32(1.0)
    lo = jnp.float32(-0.99999994)
    hi = jnp.float32(1.0)
    u = jnp.maximum(lo, flt * (hi - lo) + lo)
    return jnp.float32(1.4142135623730951) * lax.erf_inv(u)


def _router_body(rows_per_blk, kd_ref, x_ref, xout_ref,
                 probs_ref, topv_ref, topi_ref):
    i = pl.program_id(0)
    k0 = kd_ref[0]
    k1 = kd_ref[1]
    xout_ref[...] = x_ref[...]
    base = (i * (rows_per_blk * NUM_EXPERTS)).astype(_U32)
    row = lax.broadcasted_iota(_U32, (rows_per_blk, NUM_EXPERTS), 0)
    col = lax.broadcasted_iota(_U32, (rows_per_blk, NUM_EXPERTS), 1)
    cnt = base + row * _U32(NUM_EXPERTS) + col
    bits = _threefry_xored(k0, k1, cnt)
    probs = _bits_to_normal(bits)
    probs_ref[...] = probs

    b = lax.bitcast_convert_type(probs, _U32)
    sortable = jnp.where((b >> _U32(31)) != 0, ~b, b | _U32(0x80000000))
    colu = lax.broadcasted_iota(_U32, (rows_per_blk, NUM_EXPERTS), 1)
    keyp = lax.bitcast_convert_type(
        ((sortable & _U32(0xFFFFFFC0)) | (_U32(63) - colu)) ^ _U32(0x80000000),
        jnp.int32,
    )
    for j in range(TOP_K):
        ms = jnp.max(keyp, axis=1, keepdims=True)
        m = lax.bitcast_convert_type(ms, _U32) ^ _U32(0x80000000)
        idx = (_U32(63) - (m & _U32(63))).astype(jnp.int32)
        st = m & _U32(0xFFFFFFC0)
        vb = jnp.where((st >> _U32(31)) != 0, st ^ _U32(0x80000000), ~st)
        topv_ref[:, j : j + 1] = lax.bitcast_convert_type(vb, jnp.float32)
        topi_ref[:, j : j + 1] = idx
        keyp = jnp.where(keyp == ms, jnp.int32(-(2**31)), keyp)


def kernel(x, key):
    kd = jax.random.key_data(key).astype(jnp.uint32)
    rows_per_blk = 512
    grid = (SEQ // rows_per_blk,)
    d_model = x.shape[1]
    xout, probs, topv, topi = pl.pallas_call(
        functools.partial(_router_body, rows_per_blk),
        grid=grid,
        in_specs=[
            pl.BlockSpec(memory_space=pltpu.SMEM),
            pl.BlockSpec((rows_per_blk, d_model), lambda i: (i, 0)),
        ],
        out_specs=[
            pl.BlockSpec((rows_per_blk, d_model), lambda i: (i, 0)),
            pl.BlockSpec((rows_per_blk, NUM_EXPERTS), lambda i: (i, 0)),
            pl.BlockSpec((rows_per_blk, TOP_K), lambda i: (i, 0)),
            pl.BlockSpec((rows_per_blk, TOP_K), lambda i: (i, 0)),
        ],
        out_shape=[
            jax.ShapeDtypeStruct(x.shape, x.dtype),
            jax.ShapeDtypeStruct((SEQ, NUM_EXPERTS), jnp.float32),
            jax.ShapeDtypeStruct((SEQ, TOP_K), jnp.float32),
            jax.ShapeDtypeStruct((SEQ, TOP_K), jnp.int32),
        ],
        compiler_params=pltpu.CompilerParams(
            dimension_semantics=("arbitrary",),
        ),
    )(kd, x)
    return (xout, probs, topv, topi)
